# Initial kernel scaffold; baseline (speedup 1.0000x reference)
import jax
import jax.numpy as jnp
from jax.experimental import pallas as pl


def kernel(x, mapping, memory):
    raise NotImplementedError("write your pallas kernel here")


# speedup vs baseline: 51.6896x; 51.6896x over previous
import functools

import jax
import jax.numpy as jnp
from jax import lax
from jax.experimental import pallas as pl
from jax.experimental.pallas import tpu as pltpu
from jax.experimental.pallas import tpu_sc as plsc

INPUT_BITS = 16384
OUTPUT_BITS = 65536
N_BITS = 8
N_CELLS = 2 ** N_BITS

_NW = 32
_P = OUTPUT_BITS // _NW
_L = 16
_CHUNK = 128


def _tile_body(x_hbm, map_hbm, mem_hbm, out_hbm, x_v, m_v, idx_v, o_v, sem):
    nc = 2
    wid = lax.axis_index("s") * nc + lax.axis_index("c")
    base = wid * _P

    pltpu.sync_copy(x_hbm, x_v)
    pltpu.sync_copy(map_hbm.at[:, pl.ds(base, _P)], m_v)

    lane = lax.iota(jnp.int32, 16)
    row0 = (base + lane) * N_CELLS

    def body(i, _):
        off = i * _L
        acc = jnp.zeros((16,), jnp.int32)
        for j in range(N_BITS):
            bit_idx = m_v[j, pl.ds(off, _L)]
            bits = plsc.load_gather(x_v, [bit_idx])
            acc = acc + (bits << j)
        idx_v[pl.ds(off, _L)] = row0 + off * N_CELLS + acc
        return 0

    lax.fori_loop(0, _P // _L, body, 0, unroll=4)

    copies = []
    for c in range(_P // _CHUNK):
        copies.append(
            pltpu.async_copy(
                mem_hbm.at[idx_v.at[pl.ds(c * _CHUNK, _CHUNK)]],
                o_v.at[pl.ds(c * _CHUNK, _CHUNK)],
                sem,
            )
        )
    for cp in copies:
        cp.wait()

    pltpu.sync_copy(o_v, out_hbm.at[pl.ds(base, _P)])


@jax.jit
def _run(x, mapping_t, mem_flat):
    mesh = plsc.VectorSubcoreMesh(core_axis_name="c", subcore_axis_name="s")
    fn = pl.kernel(
        _tile_body,
        out_type=jax.ShapeDtypeStruct((OUTPUT_BITS,), jnp.float32),
        mesh=mesh,
        scratch_types=[
            pltpu.VMEM((INPUT_BITS,), jnp.int32),
            pltpu.VMEM((N_BITS, _P), jnp.int32),
            pltpu.VMEM((_P,), jnp.int32),
            pltpu.VMEM((_P,), jnp.float32),
            pltpu.SemaphoreType.DMA,
        ],
        compiler_params=pltpu.CompilerParams(needs_layout_passes=False),
    )
    return fn(x, mapping_t, mem_flat)


def kernel(x, mapping, memory):
    mapping_t = mapping.T.reshape(N_BITS, OUTPUT_BITS)
    mem_flat = memory.reshape(OUTPUT_BITS * N_CELLS)
    return _run(x, mapping_t, mem_flat)
